# baseline (speedup 1.0000x reference)
import jax
import jax.numpy as jnp
from jax import lax
from jax.experimental import pallas as pl
from jax.experimental.pallas import tpu as pltpu
from jax.experimental.pallas import tpu_sc as plsc

_BATCH = 4096
_NN = 16384
_SY = 8192
_NW = 32
_R = _BATCH // _NW
_D = 4


def _sc_body(act, comb, out, comb_v, in0, in1, in2, in3, out0, out1, out2, out3,
             si0, si1, si2, si3, so0, so1, so2, so3):
    c = lax.axis_index("c")
    s = lax.axis_index("s")
    wid = s * 2 + c
    rowbase = wid * _R

    pltpu.sync_copy(comb, comb_v)

    ins = (in0, in1, in2, in3)
    outs = (out0, out1, out2, out3)
    sis = (si0, si1, si2, si3)
    sos = (so0, so1, so2, so3)

    def in_copy(r, b):
        return pltpu.make_async_copy(act.at[rowbase + r], ins[b], sis[b])

    def out_copy(r, b):
        return pltpu.make_async_copy(outs[b], out.at[rowbase + r], sos[b])

    def compute(b):
        inb = ins[b]
        outb = outs[b]

        @plsc.parallel_loop(0, _SY // 16, unroll=16)
        def _(j):
            cv = comb_v[pl.ds(j * 16, 16)]
            il = cv & 0xFFFF
            ir = cv >> 16
            a = plsc.load_gather(inb, [il])
            bb = plsc.load_gather(inb, [ir])
            outb[pl.ds(j * 16, 16)] = a * bb

    for b in range(_D):
        in_copy(b, b).start()

    for b in range(_D):
        in_copy(b, b).wait()
        compute(b)
        out_copy(b, b).start()
        in_copy(b + _D, b).start()

    def outer(rr, carry):
        for b in range(_D):
            r = _D * rr + b
            in_copy(r, b).wait()
            out_copy(r - _D, b).wait()
            compute(b)
            out_copy(r, b).start()
            in_copy(r + _D, b).start()
        return carry

    lax.fori_loop(1, _R // _D - 1, outer, None)

    for b in range(_D):
        r = _R - _D + b
        in_copy(r, b).wait()
        out_copy(r - _D, b).wait()
        compute(b)
        out_copy(r, b).start()
    for b in range(_D):
        out_copy(_R - _D + b, b).wait()


def kernel(post_activations, left_indices, right_indices):
    li = left_indices.astype(jnp.int32)
    ri = right_indices.astype(jnp.int32)
    comb = li | (ri << 16)

    mesh = plsc.VectorSubcoreMesh(core_axis_name="c", subcore_axis_name="s")
    f = pl.kernel(
        _sc_body,
        out_type=jax.ShapeDtypeStruct((_BATCH, _SY), jnp.float32),
        mesh=mesh,
        scratch_types=[
            pltpu.VMEM((_SY,), jnp.int32),
            pltpu.VMEM((_NN,), jnp.float32),
            pltpu.VMEM((_NN,), jnp.float32),
            pltpu.VMEM((_NN,), jnp.float32),
            pltpu.VMEM((_NN,), jnp.float32),
            pltpu.VMEM((_SY,), jnp.float32),
            pltpu.VMEM((_SY,), jnp.float32),
            pltpu.VMEM((_SY,), jnp.float32),
            pltpu.VMEM((_SY,), jnp.float32),
            pltpu.SemaphoreType.DMA,
            pltpu.SemaphoreType.DMA,
            pltpu.SemaphoreType.DMA,
            pltpu.SemaphoreType.DMA,
            pltpu.SemaphoreType.DMA,
            pltpu.SemaphoreType.DMA,
            pltpu.SemaphoreType.DMA,
            pltpu.SemaphoreType.DMA,
        ],
        compiler_params=pltpu.CompilerParams(needs_layout_passes=False),
    )
    return f(post_activations, comb)
